# baseline (speedup 1.0000x reference)
import functools

import jax
import jax.numpy as jnp
from jax import lax
from jax.experimental import pallas as pl
from jax.experimental.pallas import tpu as pltpu
from jax.experimental.pallas import tpu_sc as plsc

VOCAB = 1000000
DIM = 64
CTX = 200
BATCH = 4096

NC = 2
NS = 16
NW = NC * NS
LANES = 16

ROWS = BATCH * CTX
ROWS_PER_W = ROWS // NW
CHUNK = CTX
NCHUNK = ROWS_PER_W // CHUNK
HALF = CHUNK // 2


def _emb_kernel(tok_hbm, idx_hbm, pos_hbm, out_hbm, idx_v, pos_v, rows_v, sem):
    wid = lax.axis_index("s") * NC + lax.axis_index("c")
    base = wid * ROWS_PER_W

    pltpu.sync_copy(idx_hbm.at[wid], idx_v)
    pltpu.sync_copy(pos_hbm, pos_v)

    def chunk_body(c, carry):
        h0 = pltpu.async_copy(tok_hbm.at[idx_v.at[c, 0]],
                              rows_v.at[pl.ds(0, HALF)], sem)
        h1 = pltpu.async_copy(tok_hbm.at[idx_v.at[c, 1]],
                              rows_v.at[pl.ds(HALF, HALF)], sem)
        h0.wait()
        h1.wait()

        def add_body(r, carry2):
            for d in range(DIM // LANES):
                sl = pl.ds(d * LANES, LANES)
                plsc.addupdate(rows_v.at[r, sl], pos_v[r, sl])
            return carry2

        lax.fori_loop(0, CHUNK, add_body, 0, unroll=2)

        pltpu.sync_copy(rows_v, out_hbm.at[pl.ds(base + c * CHUNK, CHUNK)])
        return carry

    lax.fori_loop(0, NCHUNK, chunk_body, 0)


@jax.jit
def _run(token_ids, tok_table, pos_table):
    idx = token_ids.reshape(NW, NCHUNK, 2, HALF).astype(jnp.int32)
    mesh = plsc.VectorSubcoreMesh(core_axis_name="c", subcore_axis_name="s")
    out = pl.kernel(
        _emb_kernel,
        mesh=mesh,
        out_type=jax.ShapeDtypeStruct((ROWS, DIM), jnp.float32),
        scratch_types=[
            pltpu.VMEM((NCHUNK, 2, HALF), jnp.int32),
            pltpu.VMEM((CTX, DIM), jnp.float32),
            pltpu.VMEM((CHUNK, DIM), jnp.float32),
            pltpu.SemaphoreType.DMA,
        ],
        compiler_params=pltpu.CompilerParams(use_tc_tiling_on_sc=False),
    )(tok_table, idx, pos_table)
    return out.reshape(BATCH, CTX, DIM)


def kernel(token_ids, tok_table, pos_table):
    return _run(token_ids, tok_table, pos_table)


# speedup vs baseline: 1.1235x; 1.1235x over previous
import jax
import jax.numpy as jnp
from jax import lax
from jax.experimental import pallas as pl
from jax.experimental.pallas import tpu as pltpu
from jax.experimental.pallas import tpu_sc as plsc

VOCAB = 1000000
DIM = 64
CTX = 200
BATCH = 4096

NC = 2
NS = 16
NW = NC * NS
LANES = 16

ROWS = BATCH * CTX
ROWS_PER_W = ROWS // NW
CHUNK = CTX
NCHUNK = ROWS_PER_W // CHUNK
HALF = CHUNK // 2
NBUF = 4


def _emb_kernel(tok_hbm, idx_hbm, pos_hbm, out_hbm, idx_v, pos_v, rows_v,
                sin, sout):
    wid = lax.axis_index("s") * NC + lax.axis_index("c")
    base = wid * ROWS_PER_W

    pltpu.sync_copy(idx_hbm.at[wid], idx_v)
    pltpu.sync_copy(pos_hbm, pos_v)

    def gather(c, b):
        pltpu.async_copy(tok_hbm.at[idx_v.at[c, 0]],
                         rows_v.at[b, pl.ds(0, HALF)], sin[b])
        pltpu.async_copy(tok_hbm.at[idx_v.at[c, 1]],
                         rows_v.at[b, pl.ds(HALF, HALF)], sin[b])

    def wait_gather(c, b):
        pltpu.make_async_copy(tok_hbm.at[idx_v.at[c, 0]],
                              rows_v.at[b, pl.ds(0, HALF)], sin[b]).wait()
        pltpu.make_async_copy(tok_hbm.at[idx_v.at[c, 1]],
                              rows_v.at[b, pl.ds(HALF, HALF)], sin[b]).wait()

    def scatter(c, b):
        pltpu.async_copy(rows_v.at[b],
                         out_hbm.at[pl.ds(base + c * CHUNK, CHUNK)], sout[b])

    def wait_scatter(c, b):
        pltpu.make_async_copy(rows_v.at[b],
                              out_hbm.at[pl.ds(base + c * CHUNK, CHUNK)],
                              sout[b]).wait()

    gather(0, 0)
    gather(1, 1)

    def macro_body(i, carry):
        for b in range(NBUF):
            c = i * NBUF + b
            wait_gather(c, b)
            bn = (b + 2) % NBUF

            @pl.when(c + 2 < NCHUNK)
            def _():
                gather(c + 2, bn)

            def add_body(r, carry2):
                for u in range(4):
                    ru = r * 4 + u
                    for d in range(DIM // LANES):
                        sl = pl.ds(d * LANES, LANES)
                        plsc.addupdate(rows_v.at[b, ru, sl], pos_v[ru, sl])
                return carry2

            lax.fori_loop(0, CHUNK // 4, add_body, 0)
            scatter(c, b)
            wait_scatter(c, b)
        return carry

    lax.fori_loop(0, NCHUNK // NBUF, macro_body, 0)


@jax.jit
def _run(token_ids, tok_table, pos_table):
    idx = token_ids.reshape(NW, NCHUNK, 2, HALF).astype(jnp.int32)
    mesh = plsc.VectorSubcoreMesh(core_axis_name="c", subcore_axis_name="s")
    out = pl.kernel(
        _emb_kernel,
        mesh=mesh,
        out_type=jax.ShapeDtypeStruct((ROWS, DIM), jnp.float32),
        scratch_types=[
            pltpu.VMEM((NCHUNK, 2, HALF), jnp.int32),
            pltpu.VMEM((CTX, DIM), jnp.float32),
            pltpu.VMEM((NBUF, CHUNK, DIM), jnp.float32),
            [pltpu.SemaphoreType.DMA] * NBUF,
            [pltpu.SemaphoreType.DMA] * NBUF,
        ],
        compiler_params=pltpu.CompilerParams(use_tc_tiling_on_sc=False),
    )(tok_table, idx, pos_table)
    return out.reshape(BATCH, CTX, DIM)


def kernel(token_ids, tok_table, pos_table):
    return _run(token_ids, tok_table, pos_table)


# speedup vs baseline: 1.1287x; 1.0047x over previous
import jax
import jax.numpy as jnp
from jax import lax
from jax.experimental import pallas as pl
from jax.experimental.pallas import tpu as pltpu
from jax.experimental.pallas import tpu_sc as plsc

VOCAB = 1000000
DIM = 64
CTX = 200
BATCH = 4096

NC = 2
NS = 16
NW = NC * NS
LANES = 16

B_PER_W = BATCH // NW
CHUNK = CTX
SEG = ((0, 104), (104, 96))
NBUF = 4


def _emb_kernel(tok_hbm, idx_hbm, pos_hbm, out_hbm, idx_v, pos_v, rows_v,
                sin, sout):
    wid = lax.axis_index("s") * NC + lax.axis_index("c")
    bbase = wid * B_PER_W

    pltpu.sync_copy(idx_hbm.at[pl.ds(bbase, B_PER_W)], idx_v)
    pltpu.sync_copy(pos_hbm, pos_v)

    def gather(c, b):
        for o, n in SEG:
            pltpu.async_copy(tok_hbm.at[idx_v.at[c, pl.ds(o, n)]],
                             rows_v.at[b, pl.ds(o, n)], sin[b])

    def wait_gather(c, b):
        for o, n in SEG:
            pltpu.make_async_copy(tok_hbm.at[idx_v.at[c, pl.ds(o, n)]],
                                  rows_v.at[b, pl.ds(o, n)], sin[b]).wait()

    def scatter(c, b):
        pltpu.async_copy(rows_v.at[b], out_hbm.at[bbase + c], sout[b])

    def wait_scatter(c, b):
        pltpu.make_async_copy(rows_v.at[b], out_hbm.at[bbase + c],
                              sout[b]).wait()

    gather(0, 0)
    gather(1, 1)

    def macro_body(i, carry):
        for b in range(NBUF):
            c = i * NBUF + b
            wait_gather(c, b)
            bn = (b + 2) % NBUF

            @pl.when(c + 2 < B_PER_W)
            def _():
                gather(c + 2, bn)

            def add_body(r, carry2):
                for u in range(4):
                    ru = r * 4 + u
                    for d in range(DIM // LANES):
                        sl = pl.ds(d * LANES, LANES)
                        plsc.addupdate(rows_v.at[b, ru, sl], pos_v[ru, sl])
                return carry2

            lax.fori_loop(0, CHUNK // 4, add_body, 0)
            scatter(c, b)
            wait_scatter(c, b)
        return carry

    lax.fori_loop(0, B_PER_W // NBUF, macro_body, 0)


@jax.jit
def _run(token_ids, tok_table, pos_table):
    mesh = plsc.VectorSubcoreMesh(core_axis_name="c", subcore_axis_name="s")
    return pl.kernel(
        _emb_kernel,
        mesh=mesh,
        out_type=jax.ShapeDtypeStruct((BATCH, CTX, DIM), jnp.float32),
        scratch_types=[
            pltpu.VMEM((B_PER_W, CTX), jnp.int32),
            pltpu.VMEM((CTX, DIM), jnp.float32),
            pltpu.VMEM((NBUF, CHUNK, DIM), jnp.float32),
            [pltpu.SemaphoreType.DMA] * NBUF,
            [pltpu.SemaphoreType.DMA] * NBUF,
        ],
        compiler_params=pltpu.CompilerParams(use_tc_tiling_on_sc=False),
    )(tok_table, token_ids.astype(jnp.int32), pos_table)


def kernel(token_ids, tok_table, pos_table):
    return _run(token_ids, tok_table, pos_table)
